# baseline (speedup 1.0000x reference)
import functools

import jax
import jax.numpy as jnp
from jax import lax
from jax.experimental import pallas as pl
from jax.experimental.pallas import tpu as pltpu
from jax.experimental.pallas import tpu_sc as plsc

_BATCH = 4
_SEQ = 4096
_DIM = 1024
_NC = 2
_NS = 16
_NW = _NC * _NS
_ROWS_PER_W = _SEQ // _NW
_CHUNK = 32
_NCHUNK = _ROWS_PER_W // _CHUNK

_TC_BATCH = 3
_SBLK = 2048


def _tc_body(w_ref, o_ref):
    o_ref[...] = w_ref[...][None]


@functools.partial(
    pl.kernel,
    mesh=plsc.VectorSubcoreMesh(core_axis_name="c", subcore_axis_name="s"),
    out_type=jax.ShapeDtypeStruct((_SEQ, _DIM), jnp.float32),
    scratch_types=[
        pltpu.VMEM((_CHUNK, _DIM), jnp.float32),
        pltpu.VMEM((_CHUNK, _DIM), jnp.float32),
        pltpu.SemaphoreType.DMA,
        pltpu.SemaphoreType.DMA,
    ],
)
def _sc_one_batch(w_hbm, out_hbm, buf_a, buf_b, wsem_a, wsem_b):
    wid = lax.axis_index("s") * _NC + lax.axis_index("c")
    base = wid * _ROWS_PER_W
    bufs = (buf_a, buf_b)
    wsems = (wsem_a, wsem_b)
    writes = []
    for i in range(_NCHUNK):
        buf = bufs[i % 2]
        wsem = wsems[i % 2]
        if i >= 2:
            writes[i - 2].wait()
        rows = pl.ds(base + i * _CHUNK, _CHUNK)
        pltpu.sync_copy(w_hbm.at[rows], buf)
        cp = pltpu.make_async_copy(buf, out_hbm.at[rows], wsem)
        cp.start()
        writes.append(cp)
    writes[_NCHUNK - 2].wait()
    writes[_NCHUNK - 1].wait()


def kernel(input, weights):
    del input
    out_sc = _sc_one_batch(weights)
    out_tc = pl.pallas_call(
        _tc_body,
        grid=(_SEQ // _SBLK, _TC_BATCH),
        in_specs=[pl.BlockSpec((_SBLK, _DIM), lambda i, b: (i, 0))],
        out_specs=pl.BlockSpec((1, _SBLK, _DIM), lambda i, b: (b, i, 0)),
        out_shape=jax.ShapeDtypeStruct((_TC_BATCH, _SEQ, _DIM), jnp.float32),
    )(weights)
    return jnp.concatenate([out_tc, out_sc[None]], axis=0)


# speedup vs baseline: 2.0027x; 2.0027x over previous
import functools

import jax
import jax.numpy as jnp
from jax import lax
from jax.experimental import pallas as pl
from jax.experimental.pallas import tpu as pltpu
from jax.experimental.pallas import tpu_sc as plsc

_BATCH = 4
_SEQ = 4096
_DIM = 1024
_NC = 2
_NS = 16
_NW = _NC * _NS
_ROWS_PER_W = _SEQ // _NW
_CHUNK = 32
_NCHUNK = _ROWS_PER_W // _CHUNK
_NBUF = 3


@functools.partial(
    pl.kernel,
    mesh=plsc.VectorSubcoreMesh(core_axis_name="c", subcore_axis_name="s"),
    out_type=jax.ShapeDtypeStruct((_BATCH, _SEQ, _DIM), jnp.float32),
    scratch_types=[
        pltpu.VMEM((_NBUF, _CHUNK, _DIM), jnp.float32),
        pltpu.SemaphoreType.DMA,
        pltpu.SemaphoreType.DMA,
        pltpu.SemaphoreType.DMA,
        pltpu.SemaphoreType.DMA,
    ],
)
def _broadcast_rows(w_hbm, out_hbm, ring, rsem, wsem_0, wsem_1, wsem_2):
    wid = lax.axis_index("s") * _NC + lax.axis_index("c")
    base = wid * _ROWS_PER_W
    wsems = (wsem_0, wsem_1, wsem_2)

    def row_slice(i):
        return pl.ds(base + i * _CHUNK, _CHUNK)

    reads = []
    cp = pltpu.make_async_copy(w_hbm.at[row_slice(0)], ring.at[0], rsem)
    cp.start()
    reads.append(cp)

    writes = []
    for i in range(_NCHUNK):
        slot = i % _NBUF
        nxt = i + 1
        if nxt < _NCHUNK:
            if nxt >= _NBUF:
                for cp in writes[nxt - _NBUF]:
                    cp.wait()
            cp = pltpu.make_async_copy(
                w_hbm.at[row_slice(nxt)], ring.at[nxt % _NBUF], rsem
            )
            cp.start()
            reads.append(cp)
        reads[i].wait()
        cps = [
            pltpu.make_async_copy(ring.at[slot], out_hbm.at[b].at[row_slice(i)], wsems[slot])
            for b in range(_BATCH)
        ]
        for cp in cps:
            cp.start()
        writes.append(cps)
    for i in range(max(0, _NCHUNK - _NBUF), _NCHUNK):
        for cp in writes[i]:
            cp.wait()


def kernel(input, weights):
    del input
    return _broadcast_rows(weights)
